# baseline (speedup 1.0000x reference)
import jax
import jax.numpy as jnp
from jax.experimental import pallas as pl
from jax.experimental.pallas import tpu as pltpu

E = 8
H = 32
DIN = 3
P = E * H
K2 = P + E

_THRESH = -(2.0 ** -25)


def _fwd_kernel(
    xt_ref, w1_ref, b1_ref, w2_ref, b2_ref, w3_ref, b3_ref, o_ref,
):
    bf16 = jnp.bfloat16
    f32 = jnp.float32

    w1 = w1_ref[...]
    tab1 = jnp.concatenate(
        [jnp.transpose(w1[:, d, :]) for d in range(DIN)]
        + [jnp.transpose(b1_ref[...])],
        axis=1,
    ).astype(bf16)
    w2 = w2_ref[...]
    tab2 = jnp.concatenate(
        [jnp.transpose(w2[e]) for e in range(E)]
        + [jnp.transpose(b2_ref[...])],
        axis=1,
    ).astype(bf16)

    xt = xt_ref[...]
    B = xt.shape[1]
    bits = jnp.where(xt >= _THRESH, 1.0, 0.0)
    mid = bits[0:1, :] + 2.0 * bits[1:2, :] + 4.0 * bits[2:3, :]
    midi = mid.astype(jnp.int32)
    mid8 = jnp.broadcast_to(midi, (E, B))
    eids8 = jax.lax.broadcasted_iota(jnp.int32, (E, B), 0)
    oh8 = jnp.where(mid8 == eids8, 1.0, 0.0)

    x24 = jnp.concatenate(
        [jnp.broadcast_to(xt[d : d + 1, :], (E, B)) for d in range(DIN)], axis=0
    )
    oh24 = jnp.concatenate([oh8] * DIN, axis=0)
    xk = jnp.concatenate([x24 * oh24, oh8], axis=0).astype(bf16)
    a1 = jax.lax.dot_general(
        tab1, xk, (((1,), (0,)), ((), ())), preferred_element_type=f32
    )
    t1 = jnp.tanh(a1).astype(bf16)

    oh8b = oh8.astype(bf16)
    t1rep = jnp.broadcast_to(t1[None], (E, H, B)).reshape(P, B)
    oh256 = jnp.broadcast_to(oh8b[:, None, :], (E, H, B)).reshape(P, B)
    featm = t1rep * oh256
    feat = jnp.concatenate([featm, oh8b], axis=0)
    a2 = jax.lax.dot_general(
        tab2, feat, (((1,), (0,)), ((), ())), preferred_element_type=f32
    )
    t2 = jnp.tanh(a2).astype(bf16)

    w3flat = w3_ref[...].reshape(E, H).astype(bf16)
    y8 = (
        jax.lax.dot_general(
            w3flat, t2, (((1,), (0,)), ((), ())), preferred_element_type=f32
        )
        + b3_ref[...]
    )
    ysel = jnp.where(mid8 == eids8, y8, 0.0)
    o_ref[...] = jnp.sum(ysel, axis=0, keepdims=True)[None]


def kernel(x, W1, b1, W2, b2, W3, b3):
    n = x.shape[0]
    B = 16384
    nb = n // B
    xt = jnp.transpose(x)
    out = pl.pallas_call(
        _fwd_kernel,
        grid=(nb,),
        in_specs=[
            pl.BlockSpec((DIN, B), lambda i: (0, i)),
            pl.BlockSpec((E, DIN, H), lambda i: (0, 0, 0)),
            pl.BlockSpec((E, H), lambda i: (0, 0)),
            pl.BlockSpec((E, H, H), lambda i: (0, 0, 0)),
            pl.BlockSpec((E, H), lambda i: (0, 0)),
            pl.BlockSpec((E, H, 1), lambda i: (0, 0, 0)),
            pl.BlockSpec((E, 1), lambda i: (0, 0)),
        ],
        out_specs=pl.BlockSpec((1, 1, B), lambda i: (i, 0, 0)),
        out_shape=jax.ShapeDtypeStruct((nb, 1, B), jnp.float32),
        compiler_params=pltpu.CompilerParams(
            dimension_semantics=("parallel",),
        ),
    )(xt, W1, b1, W2, b2, W3, b3)
    return out.reshape(n, 1)


# speedup vs baseline: 1.0464x; 1.0464x over previous
import jax
import jax.numpy as jnp
from jax.experimental import pallas as pl
from jax.experimental.pallas import tpu as pltpu

E = 8
H = 32
DIN = 3
P = E * H
K2 = P + E

_THRESH = -(2.0 ** -25)


def _fwd_kernel(
    xt_ref, w1_ref, b1_ref, w2_ref, b2_ref, w3_ref, b3_ref, o_ref,
    tab1_s, tab2_s,
):
    bf16 = jnp.bfloat16
    f32 = jnp.float32

    @pl.when(pl.program_id(0) == 0)
    def _init():
        w1 = w1_ref[...]
        tab1_s[...] = jnp.concatenate(
            [jnp.transpose(w1[:, d, :]) for d in range(DIN)]
            + [jnp.transpose(b1_ref[...])],
            axis=1,
        ).astype(bf16)
        w2 = w2_ref[...]
        tab2_s[...] = jnp.concatenate(
            [jnp.transpose(w2[e]) for e in range(E)]
            + [jnp.transpose(b2_ref[...])],
            axis=1,
        ).astype(bf16)

    tab1 = tab1_s[...]
    tab2 = tab2_s[...]

    xt = xt_ref[...]
    B = xt.shape[1]
    bits = jnp.where(xt >= _THRESH, 1.0, 0.0)
    mid = bits[0:1, :] + 2.0 * bits[1:2, :] + 4.0 * bits[2:3, :]
    midi = mid.astype(jnp.int32)
    mid8 = jnp.broadcast_to(midi, (E, B))
    eids8 = jax.lax.broadcasted_iota(jnp.int32, (E, B), 0)
    oh8 = jnp.where(mid8 == eids8, 1.0, 0.0)

    x24 = jnp.concatenate(
        [jnp.broadcast_to(xt[d : d + 1, :], (E, B)) for d in range(DIN)], axis=0
    )
    oh24 = jnp.concatenate([oh8] * DIN, axis=0)
    xk = jnp.concatenate([x24 * oh24, oh8], axis=0).astype(bf16)
    a1 = jax.lax.dot_general(
        tab1, xk, (((1,), (0,)), ((), ())), preferred_element_type=f32
    )
    t1 = jnp.tanh(a1).astype(bf16)

    oh8b = oh8.astype(bf16)
    t1rep = jnp.broadcast_to(t1[None], (E, H, B)).reshape(P, B)
    oh256 = jnp.broadcast_to(oh8b[:, None, :], (E, H, B)).reshape(P, B)
    featm = t1rep * oh256
    feat = jnp.concatenate([featm, oh8b], axis=0)
    a2 = jax.lax.dot_general(
        tab2, feat, (((1,), (0,)), ((), ())), preferred_element_type=f32
    )
    t2 = jnp.tanh(a2).astype(bf16)

    w3flat = w3_ref[...].reshape(E, H).astype(bf16)
    y8 = (
        jax.lax.dot_general(
            w3flat, t2, (((1,), (0,)), ((), ())), preferred_element_type=f32
        )
        + b3_ref[...]
    )
    ysel = jnp.where(mid8 == eids8, y8, 0.0)
    o_ref[...] = jnp.sum(ysel, axis=0, keepdims=True)[None]


def kernel(x, W1, b1, W2, b2, W3, b3):
    n = x.shape[0]
    B = 16384
    nb = n // B
    xt = jnp.transpose(x)
    out = pl.pallas_call(
        _fwd_kernel,
        grid=(nb,),
        in_specs=[
            pl.BlockSpec((DIN, B), lambda i: (0, i)),
            pl.BlockSpec((E, DIN, H), lambda i: (0, 0, 0)),
            pl.BlockSpec((E, H), lambda i: (0, 0)),
            pl.BlockSpec((E, H, H), lambda i: (0, 0, 0)),
            pl.BlockSpec((E, H), lambda i: (0, 0)),
            pl.BlockSpec((E, H, 1), lambda i: (0, 0, 0)),
            pl.BlockSpec((E, 1), lambda i: (0, 0)),
        ],
        out_specs=pl.BlockSpec((1, 1, B), lambda i: (i, 0, 0)),
        out_shape=jax.ShapeDtypeStruct((nb, 1, B), jnp.float32),
        scratch_shapes=[
            pltpu.VMEM((H, H), jnp.bfloat16),
            pltpu.VMEM((H, K2), jnp.bfloat16),
        ],
    )(xt, W1, b1, W2, b2, W3, b3)
    return out.reshape(n, 1)


# speedup vs baseline: 1.1096x; 1.0604x over previous
import jax
import jax.numpy as jnp
from jax.experimental import pallas as pl
from jax.experimental.pallas import tpu as pltpu

E = 8
H = 32
DIN = 3
P = E * H
K2 = P + E

_THRESH = -(2.0 ** -25)


def _fwd_kernel(
    xt_ref, w1_ref, b1_ref, w2_ref, b2_ref, w3_ref, b3_ref, o_ref,
    tab1_s, tab2_s,
):
    bf16 = jnp.bfloat16
    f32 = jnp.float32

    @pl.when(pl.program_id(0) == 0)
    def _init():
        w1 = w1_ref[...]
        tab1_s[...] = jnp.concatenate(
            [jnp.transpose(w1[:, d, :]) for d in range(DIN)]
            + [jnp.transpose(b1_ref[...])],
            axis=1,
        ).astype(bf16)
        w2 = w2_ref[...]
        tab2_s[...] = jnp.concatenate(
            [jnp.transpose(w2[e]) for e in range(E)]
            + [jnp.transpose(b2_ref[...])],
            axis=1,
        ).astype(bf16)

    tab1 = tab1_s[...]
    tab2 = tab2_s[...]

    xt = xt_ref[...]
    B = xt.shape[1]
    bits = jnp.where(xt >= _THRESH, 1.0, 0.0)
    mid = bits[0:1, :] + 2.0 * bits[1:2, :] + 4.0 * bits[2:3, :]
    midi = mid.astype(jnp.int32)
    mid8 = jnp.broadcast_to(midi, (E, B))
    eids8 = jax.lax.broadcasted_iota(jnp.int32, (E, B), 0)
    oh8 = jnp.where(mid8 == eids8, 1.0, 0.0)

    x24 = jnp.concatenate(
        [jnp.broadcast_to(xt[d : d + 1, :], (E, B)) for d in range(DIN)], axis=0
    )
    oh24 = jnp.concatenate([oh8] * DIN, axis=0)
    xk = jnp.concatenate([x24 * oh24, oh8], axis=0).astype(bf16)
    a1 = jax.lax.dot_general(
        tab1, xk, (((1,), (0,)), ((), ())), preferred_element_type=f32
    )
    t1 = jnp.tanh(a1).astype(bf16)

    oh8b = oh8.astype(bf16)
    feat = jnp.concatenate(
        [t1 * oh8b[e : e + 1, :] for e in range(E)] + [oh8b], axis=0
    )
    a2 = jax.lax.dot_general(
        tab2, feat, (((1,), (0,)), ((), ())), preferred_element_type=f32
    )
    t2 = jnp.tanh(a2).astype(bf16)

    w3flat = w3_ref[...].reshape(E, H).astype(bf16)
    y8 = (
        jax.lax.dot_general(
            w3flat, t2, (((1,), (0,)), ((), ())), preferred_element_type=f32
        )
        + b3_ref[...]
    )
    ysel = jnp.where(mid8 == eids8, y8, 0.0)
    o_ref[...] = jnp.sum(ysel, axis=0, keepdims=True)[None]


def kernel(x, W1, b1, W2, b2, W3, b3):
    n = x.shape[0]
    B = 32768
    nb = n // B
    xt = jnp.transpose(x)
    out = pl.pallas_call(
        _fwd_kernel,
        grid=(nb,),
        in_specs=[
            pl.BlockSpec((DIN, B), lambda i: (0, i)),
            pl.BlockSpec((E, DIN, H), lambda i: (0, 0, 0)),
            pl.BlockSpec((E, H), lambda i: (0, 0)),
            pl.BlockSpec((E, H, H), lambda i: (0, 0, 0)),
            pl.BlockSpec((E, H), lambda i: (0, 0)),
            pl.BlockSpec((E, H, 1), lambda i: (0, 0, 0)),
            pl.BlockSpec((E, 1), lambda i: (0, 0)),
        ],
        out_specs=pl.BlockSpec((1, 1, B), lambda i: (i, 0, 0)),
        out_shape=jax.ShapeDtypeStruct((nb, 1, B), jnp.float32),
        scratch_shapes=[
            pltpu.VMEM((H, H), jnp.bfloat16),
            pltpu.VMEM((H, K2), jnp.bfloat16),
        ],
    )(xt, W1, b1, W2, b2, W3, b3)
    return out.reshape(n, 1)
